# baseline (speedup 1.0000x reference)
import functools

import jax
import jax.numpy as jnp
from jax import lax
from jax.experimental import pallas as pl
from jax.experimental.pallas import tpu as pltpu
from jax.experimental.pallas import tpu_sc as plsc

N = 10000
E = 320000
D = 128

NC = 2
NS = 16
NW = NC * NS
NP = 10240
EPW = E // NW
CHUNK = 100
NCHUNK = EPW // CHUNK
RPT = NP // NS
ZCH = 64
ZB = RPT // ZCH
GROUP = 25
NGROUP = NCHUNK // GROUP

_MESH = plsc.VectorSubcoreMesh(core_axis_name="c", subcore_axis_name="s")


@functools.partial(
    pl.kernel, mesh=_MESH,
    out_type=jax.ShapeDtypeStruct((NC * NP, D), jnp.float32),
    scratch_types=[
        pltpu.VMEM((GROUP, CHUNK), jnp.int32),
        pltpu.VMEM((GROUP, CHUNK), jnp.int32),
        pltpu.VMEM((CHUNK, D), jnp.float32),
        pltpu.VMEM((CHUNK, D), jnp.float32),
        pltpu.VMEM_SHARED((NP, D), jnp.float32),
    ],
)
def _sc_feature_agg(feats_h, src_h, dst_h, zf_h, out_h, sidx, didx, rows,
                    rows2, acc):
    c = lax.axis_index("c")
    s = lax.axis_index("s")
    wid = c * NS + s
    base = c * NP + s * RPT
    pltpu.sync_copy(zf_h, rows)

    def zbody(j, carry):
        pltpu.sync_copy(rows.at[pl.ds(0, ZCH)],
                        acc.at[pl.ds(s * RPT + j * ZCH, ZCH)])
        return carry

    lax.fori_loop(0, ZB, zbody, 0)
    plsc.subcore_barrier()

    def _pipeline(sem_a, sem_b):
        for g in range(NGROUP):
            pltpu.sync_copy(src_h.at[wid, g], sidx)
            pltpu.sync_copy(dst_h.at[wid, g], didx)
            pltpu.async_copy(feats_h.at[sidx.at[0]], rows, sem_a)

            def ebody(k, carry):
                i = 2 * k
                pltpu.async_copy(feats_h.at[sidx.at[i + 1]], rows2, sem_b)
                pltpu.make_async_copy(feats_h.at[sidx.at[i]], rows,
                                      sem_a).wait()
                pltpu.sync_copy(rows, acc.at[didx.at[i]], add=True)
                pltpu.async_copy(feats_h.at[sidx.at[i + 2]], rows, sem_a)
                pltpu.make_async_copy(feats_h.at[sidx.at[i + 1]], rows2,
                                      sem_b).wait()
                pltpu.sync_copy(rows2, acc.at[didx.at[i + 1]], add=True)
                return carry

            lax.fori_loop(0, (GROUP - 1) // 2, ebody, 0)
            pltpu.make_async_copy(feats_h.at[sidx.at[GROUP - 1]], rows,
                                  sem_a).wait()
            pltpu.sync_copy(rows, acc.at[didx.at[GROUP - 1]], add=True)

    pl.run_scoped(_pipeline, sem_a=pltpu.SemaphoreType.DMA,
                  sem_b=pltpu.SemaphoreType.DMA)
    plsc.subcore_barrier()

    def dbody(j, carry):
        pltpu.sync_copy(acc.at[pl.ds(s * RPT + j * ZCH, ZCH)],
                        rows.at[pl.ds(0, ZCH)])
        pltpu.sync_copy(rows.at[pl.ds(0, ZCH)],
                        out_h.at[pl.ds(base + j * ZCH, ZCH)])
        return carry

    lax.fori_loop(0, ZB, dbody, 0)


def _tc_sage(partials, cnt_partials, feats, WlT, bl, WrT, relu):
    Bn = 1024
    grid = (NP // Bn,)

    def body(p_ref, c_ref, f_ref, wl_ref, b_ref, wr_ref, o_ref):
        summed = p_ref[0] + p_ref[1]
        cnt = c_ref[0] + c_ref[1]
        agg = summed / jnp.maximum(cnt, 1.0)
        r = (jnp.dot(agg, wl_ref[...], preferred_element_type=jnp.float32,
                     precision=lax.Precision.HIGHEST)
             + jnp.dot(f_ref[...], wr_ref[...],
                       preferred_element_type=jnp.float32,
                       precision=lax.Precision.HIGHEST)
             + b_ref[...])
        if relu:
            r = jnp.maximum(r, 0.0)
        o_ref[...] = r

    return pl.pallas_call(
        body,
        grid=grid,
        in_specs=[
            pl.BlockSpec((NC, Bn, D), lambda i: (0, i, 0)),
            pl.BlockSpec((NC, Bn, D), lambda i: (0, i, 0)),
            pl.BlockSpec((Bn, D), lambda i: (i, 0)),
            pl.BlockSpec((D, D), lambda i: (0, 0)),
            pl.BlockSpec((1, D), lambda i: (0, 0)),
            pl.BlockSpec((D, D), lambda i: (0, 0)),
        ],
        out_specs=pl.BlockSpec((Bn, D), lambda i: (i, 0)),
        out_shape=jax.ShapeDtypeStruct((NP, D), jnp.float32),
    )(partials, cnt_partials, feats, WlT, bl, WrT)


def kernel(x, edge_index, W1l, b1l, W1r, W2l, b2l, W2r):
    src = edge_index[0].reshape(NW, NGROUP, GROUP, CHUNK)
    dst = edge_index[1].reshape(NW, NGROUP, GROUP, CHUNK)
    x_p = jnp.pad(x, ((0, NP - N), (0, 0)))

    zeros_f = jnp.zeros((CHUNK, D), jnp.float32)
    ones_mat = jnp.ones((8, D), jnp.float32)
    src_zero = jnp.zeros_like(src)

    cntp = _sc_feature_agg(ones_mat, src_zero, dst, zeros_f).reshape(NC, NP, D)
    p1 = _sc_feature_agg(x_p, src, dst, zeros_f).reshape(NC, NP, D)
    h = _tc_sage(p1, cntp, x_p, W1l.T, b1l.reshape(1, D), W1r.T, relu=True)
    p2 = _sc_feature_agg(h, src, dst, zeros_f).reshape(NC, NP, D)
    out = _tc_sage(p2, cntp, h, W2l.T, b2l.reshape(1, D), W2r.T, relu=False)
    return out[:N]


# speedup vs baseline: 28.2974x; 28.2974x over previous
import functools

import jax
import jax.numpy as jnp
from jax import lax
from jax.experimental import pallas as pl
from jax.experimental.pallas import tpu as pltpu
from jax.experimental.pallas import tpu_sc as plsc

N = 10000
E = 320000
D = 128

NC = 2
NS = 16
NW = NC * NS
NP = 10240
EPW = E // NW
CHUNK = 100
NCHUNK = EPW // CHUNK
RPT = NP // NS
ZCH = 64
ZB = RPT // ZCH
GROUP = 25
NGROUP = NCHUNK // GROUP

_MESH = plsc.VectorSubcoreMesh(core_axis_name="c", subcore_axis_name="s")


@functools.partial(
    pl.kernel, mesh=_MESH,
    out_type=jax.ShapeDtypeStruct((NC * NP, D), jnp.float32),
    scratch_types=[
        pltpu.VMEM((GROUP, CHUNK), jnp.int32),
        pltpu.VMEM((GROUP, CHUNK), jnp.int32),
        pltpu.VMEM((CHUNK, D), jnp.float32),
        pltpu.VMEM((CHUNK, D), jnp.float32),
        pltpu.VMEM_SHARED((NP, D), jnp.float32),
    ],
)
def _sc_feature_agg(feats_h, src_h, dst_h, zf_h, out_h, sidx, didx, rows,
                    rows2, acc):
    c = lax.axis_index("c")
    s = lax.axis_index("s")
    wid = c * NS + s
    base = c * NP + s * RPT
    pltpu.sync_copy(zf_h, rows)

    def zbody(j, carry):
        pltpu.sync_copy(rows.at[pl.ds(0, ZCH)],
                        acc.at[pl.ds(s * RPT + j * ZCH, ZCH)])
        return carry

    lax.fori_loop(0, ZB, zbody, 0)
    plsc.subcore_barrier()

    def _pipeline(sem_a, sem_b):
        for g in range(NGROUP):
            pltpu.sync_copy(src_h.at[wid, g], sidx)
            pltpu.sync_copy(dst_h.at[wid, g], didx)
            pltpu.async_copy(feats_h.at[sidx.at[0]], rows, sem_a)

            def ebody(k, carry):
                i = 2 * k
                pltpu.async_copy(feats_h.at[sidx.at[i + 1]], rows2, sem_b)
                pltpu.make_async_copy(feats_h.at[sidx.at[i]], rows,
                                      sem_a).wait()
                pltpu.sync_copy(rows, acc.at[didx.at[i]], add=True)
                pltpu.async_copy(feats_h.at[sidx.at[i + 2]], rows, sem_a)
                pltpu.make_async_copy(feats_h.at[sidx.at[i + 1]], rows2,
                                      sem_b).wait()
                pltpu.sync_copy(rows2, acc.at[didx.at[i + 1]], add=True)
                return carry

            lax.fori_loop(0, (GROUP - 1) // 2, ebody, 0)
            pltpu.make_async_copy(feats_h.at[sidx.at[GROUP - 1]], rows,
                                  sem_a).wait()
            pltpu.sync_copy(rows, acc.at[didx.at[GROUP - 1]], add=True)

    pl.run_scoped(_pipeline, sem_a=pltpu.SemaphoreType.DMA,
                  sem_b=pltpu.SemaphoreType.DMA)
    plsc.subcore_barrier()

    def dbody(j, carry):
        pltpu.sync_copy(acc.at[pl.ds(s * RPT + j * ZCH, ZCH)],
                        rows.at[pl.ds(0, ZCH)])
        pltpu.sync_copy(rows.at[pl.ds(0, ZCH)],
                        out_h.at[pl.ds(base + j * ZCH, ZCH)])
        return carry

    lax.fori_loop(0, ZB, dbody, 0)


def _tc_sage(partials, cnt_partials, feats, WlT, bl, WrT, relu):
    Bn = 1024
    grid = (NP // Bn,)

    def body(p_ref, c_ref, f_ref, wl_ref, b_ref, wr_ref, o_ref):
        summed = p_ref[0] + p_ref[1]
        cnt = c_ref[0] + c_ref[1]
        agg = summed / jnp.maximum(cnt, 1.0)
        r = (jnp.dot(agg, wl_ref[...], preferred_element_type=jnp.float32,
                     precision=lax.Precision.HIGHEST)
             + jnp.dot(f_ref[...], wr_ref[...],
                       preferred_element_type=jnp.float32,
                       precision=lax.Precision.HIGHEST)
             + b_ref[...])
        if relu:
            r = jnp.maximum(r, 0.0)
        o_ref[...] = r

    return pl.pallas_call(
        body,
        grid=grid,
        in_specs=[
            pl.BlockSpec((NC, Bn, D), lambda i: (0, i, 0)),
            pl.BlockSpec((NC, Bn, D), lambda i: (0, i, 0)),
            pl.BlockSpec((Bn, D), lambda i: (i, 0)),
            pl.BlockSpec((D, D), lambda i: (0, 0)),
            pl.BlockSpec((1, D), lambda i: (0, 0)),
            pl.BlockSpec((D, D), lambda i: (0, 0)),
        ],
        out_specs=pl.BlockSpec((Bn, D), lambda i: (i, 0)),
        out_shape=jax.ShapeDtypeStruct((NP, D), jnp.float32),
    )(partials, cnt_partials, feats, WlT, bl, WrT)


def kernel(x, edge_index, W1l, b1l, W1r, W2l, b2l, W2r):
    src = edge_index[0].reshape(NW, NGROUP, GROUP, CHUNK)
    dst = edge_index[1].reshape(NW, NGROUP, GROUP, CHUNK)
    x_p = jnp.pad(x, ((0, NP - N), (0, 0)))

    zeros_f = jnp.zeros((CHUNK, D), jnp.float32)
    ones_mat = jnp.ones((NP, D), jnp.float32)

    cntp = _sc_feature_agg(ones_mat, src, dst, zeros_f).reshape(NC, NP, D)
    p1 = _sc_feature_agg(x_p, src, dst, zeros_f).reshape(NC, NP, D)
    h = _tc_sage(p1, cntp, x_p, W1l.T, b1l.reshape(1, D), W1r.T, relu=True)
    p2 = _sc_feature_agg(h, src, dst, zeros_f).reshape(NC, NP, D)
    out = _tc_sage(p2, cntp, h, W2l.T, b2l.reshape(1, D), W2r.T, relu=False)
    return out[:N]
